# Initial kernel scaffold; baseline (speedup 1.0000x reference)
import jax
import jax.numpy as jnp
from jax.experimental import pallas as pl


def kernel(boxes, scores):
    raise NotImplementedError("write your pallas kernel here")


# speedup vs baseline: 14.6138x; 14.6138x over previous
import jax
import jax.numpy as jnp
from jax.experimental import pallas as pl

_N = 20000
_R = 160
_C = 128
_P = _R * _C
_PRE_TOP_K = 1000
_KEEP = 100
_IOU_T = 0.65
_SCORE_T = 0.25


def _nms_body(boxes_ref, scores_ref, out_ref):
    scores = scores_ref[...]
    bits = jax.lax.bitcast_convert_type(scores, jnp.int32)

    def bs_body(_, carry):
        lo, hi = carry
        mid = jax.lax.div(lo + hi, jnp.int32(2))
        cnt = jnp.sum((bits >= mid).astype(jnp.int32))
        pred = cnt >= _PRE_TOP_K
        return (jnp.where(pred, mid, lo), jnp.where(pred, hi, mid))

    lo, hi = jax.lax.fori_loop(
        0, 31, bs_body, (jnp.int32(0), jnp.int32(0x3F800000))
    )
    vk = lo

    gt = bits > vk
    eq = bits == vk
    quota = jnp.int32(_PRE_TOP_K) - jnp.sum(gt.astype(jnp.int32))

    eq_f = eq.astype(jnp.float32)
    ii = jax.lax.broadcasted_iota(jnp.int32, (_C, _C), 0)
    jj = jax.lax.broadcasted_iota(jnp.int32, (_C, _C), 1)
    lt_c = (ii < jj).astype(jnp.float32)
    row_prefix = jnp.dot(eq_f, lt_c, preferred_element_type=jnp.float32)
    row_tot = jnp.sum(eq_f, axis=1)
    ri = jax.lax.broadcasted_iota(jnp.int32, (_R, _R), 0)
    rj = jax.lax.broadcasted_iota(jnp.int32, (_R, _R), 1)
    lt_r = (ri < rj).astype(jnp.float32)
    prev_rows = jnp.dot(row_tot[None, :], lt_r,
                        preferred_element_type=jnp.float32)
    rank = row_prefix + prev_rows.reshape(_R, 1)
    member = gt | (eq & (rank < quota.astype(jnp.float32)))

    ws0 = jnp.where(member & (scores > _SCORE_T), scores, -1.0)

    cx = boxes_ref[0] * 640.0
    cy = boxes_ref[1] * 640.0
    w = boxes_ref[2] * 100.0 + 1.0
    h = boxes_ref[3] * 100.0 + 1.0
    x1 = cx - w * 0.5
    y1 = cy - h * 0.5
    x2 = cx + w * 0.5
    y2 = cy + h * 0.5
    areas = (x2 - x1) * (y2 - y1)

    idx = (jax.lax.broadcasted_iota(jnp.int32, (_R, _C), 0) * _C
           + jax.lax.broadcasted_iota(jnp.int32, (_R, _C), 1))
    li = jax.lax.broadcasted_iota(jnp.int32, (1, _C), 1)

    def step(i, ws):
        m = jnp.max(ws)
        bidx = jnp.min(jnp.where(ws == m, idx, jnp.int32(2 ** 30)))
        bmask = idx == bidx
        bx1 = jnp.sum(jnp.where(bmask, x1, 0.0))
        by1 = jnp.sum(jnp.where(bmask, y1, 0.0))
        bx2 = jnp.sum(jnp.where(bmask, x2, 0.0))
        by2 = jnp.sum(jnp.where(bmask, y2, 0.0))

        ww = jnp.clip(jnp.minimum(bx2, x2) - jnp.maximum(bx1, x1), 0.0)
        hh = jnp.clip(jnp.minimum(by2, y2) - jnp.maximum(by1, y1), 0.0)
        inter = ww * hh
        barea = (bx2 - bx1) * (by2 - by1)
        iou = inter / (barea + areas - inter + 1e-7)
        ws = jnp.where(iou >= _IOU_T, -1.0, ws)
        ws = jnp.where(bmask, -1.0, ws)

        valid = m > 0.0

        def sel(l, v):
            return jnp.where(li == l, jnp.where(valid, v, 0.0), 0.0)

        row = sel(0, bx1) + sel(1, by1) + sel(2, bx2) + sel(3, by2) + sel(4, m)
        out_ref[pl.ds(i, 1), :] = row
        return ws

    jax.lax.fori_loop(0, _KEEP, step, ws0)


def kernel(boxes, scores):
    pad = _P - _N
    s_pad = jnp.concatenate(
        [scores, jnp.full((pad,), -1.0, jnp.float32)]).reshape(_R, _C)
    b_pad = jnp.concatenate(
        [boxes, jnp.zeros((pad, 4), jnp.float32)], axis=0
    ).T.reshape(4, _R, _C)
    out = pl.pallas_call(
        _nms_body,
        out_shape=jax.ShapeDtypeStruct((104, _C), jnp.float32),
    )(b_pad, s_pad)
    return out[:_KEEP, :5]
